# Initial kernel scaffold; baseline (speedup 1.0000x reference)
import jax
import jax.numpy as jnp
from jax.experimental import pallas as pl


def kernel(X, W_router, W_experts, b_experts):
    raise NotImplementedError("write your pallas kernel here")


# speedup vs baseline: 1.4514x; 1.4514x over previous
import functools

import jax
import jax.numpy as jnp
from jax.experimental import pallas as pl
from jax.experimental.pallas import tpu as pltpu

NUM_EXPERTS = 8
TOP_K = 2


def _moe_dense_body(x_ref, wr_ref, we_ref, be_ref, out_ref, scale_ref):
    e = pl.program_id(0)

    @pl.when(e == 0)
    def _router():
        x = x_ref[...]
        logits = jnp.dot(x, wr_ref[...], preferred_element_type=jnp.float32)
        t, ne = logits.shape
        eids = jax.lax.broadcasted_iota(jnp.int32, (t, ne), 1)
        m1 = jnp.max(logits, axis=1, keepdims=True)
        a1 = jnp.min(jnp.where(logits == m1, eids, ne), axis=1, keepdims=True)
        is1 = eids == a1
        masked = jnp.where(is1, -jnp.inf, logits)
        m2 = jnp.max(masked, axis=1, keepdims=True)
        a2 = jnp.min(jnp.where(masked == m2, eids, ne), axis=1, keepdims=True)
        is2 = eids == a2
        e2 = jnp.exp(m2 - m1)
        p1 = 1.0 / (1.0 + e2)
        p2 = 1.0 - p1
        scale_ref[...] = jnp.where(is1, p1, 0.0) + jnp.where(is2, p2, 0.0)

    xb = x_ref[...].astype(jnp.bfloat16)
    w = we_ref[0]
    acc = jnp.dot(xb, w, preferred_element_type=jnp.float32)
    acc = acc + be_ref[0]
    sc = scale_ref[...]
    ecol = jax.lax.broadcasted_iota(jnp.int32, sc.shape, 1)
    scol = jnp.sum(jnp.where(ecol == e, sc, 0.0), axis=1, keepdims=True)
    contrib = acc * scol

    @pl.when(e == 0)
    def _init():
        out_ref[...] = contrib

    @pl.when(e > 0)
    def _acc():
        out_ref[...] += contrib


@jax.jit
def kernel(X, W_router, W_experts, b_experts):
    B, T, D = X.shape
    x2 = X.reshape(T, D)
    we_bf = W_experts.astype(jnp.bfloat16)
    out = pl.pallas_call(
        _moe_dense_body,
        grid=(NUM_EXPERTS,),
        in_specs=[
            pl.BlockSpec((T, D), lambda e: (0, 0)),
            pl.BlockSpec((D, NUM_EXPERTS), lambda e: (0, 0)),
            pl.BlockSpec((1, D, D), lambda e: (e, 0, 0)),
            pl.BlockSpec((1, 1, D), lambda e: (e, 0, 0)),
        ],
        out_specs=pl.BlockSpec((T, D), lambda e: (0, 0)),
        out_shape=jax.ShapeDtypeStruct((T, D), jnp.float32),
        scratch_shapes=[pltpu.VMEM((T, NUM_EXPERTS), jnp.float32)],
    )(x2, W_router, we_bf, b_experts.reshape(NUM_EXPERTS, 1, D))
    return out.reshape(B, T, D)
